# Initial kernel scaffold; baseline (speedup 1.0000x reference)
import jax
import jax.numpy as jnp
from jax.experimental import pallas as pl


def kernel(x, edge_index, edge_attr, W, att_src, att_dst, bias, bn_weight, bn_bias):
    raise NotImplementedError("write your pallas kernel here")


# speedup vs baseline: 26.0851x; 26.0851x over previous
import functools

import jax
import jax.numpy as jnp
from jax import lax
from jax.experimental import pallas as pl
from jax.experimental.pallas import tpu as pltpu, tpu_sc as plsc

N = 10000
D = 128
DH = D // 2
E = 320000
E_TOTAL = E + N
NT = 16
K = 128
C = (E_TOTAL + NT * K - 1) // (NT * K)
E_PAD = NT * C * K
PER_TILE = C * K
EPS = 1e-5

NP = 10240
ROWS_PER_TILE = NP // NT


def _stage1_body(x_ref, w_ref, ap_ref, h2_ref, aa8_ref):
    h = jnp.dot(x_ref[...], w_ref[...], preferred_element_type=jnp.float32)
    h2_ref[0] = h[:, :DH]
    h2_ref[1] = h[:, DH:]
    aa8_ref[...] = jnp.dot(h, ap_ref[...], preferred_element_type=jnp.float32)


def _stage1(x, W, ap):
    return pl.pallas_call(
        _stage1_body,
        out_shape=[
            jax.ShapeDtypeStruct((2, N, DH), jnp.float32),
            jax.ShapeDtypeStruct((N, 8), jnp.float32),
        ],
    )(x, W, ap)


def _copy_row_range(src_ref, dst_ref, r0):
    for t in range(5):
        pltpu.sync_copy(src_ref.at[pl.ds(r0 + 128 * t, 128)],
                        dst_ref.at[pl.ds(r0 + 128 * t, 128)])


def _edge_body(h2_hbm, aa_hbm, src_hbm, dst_hbm, acc_out, den_out,
               src_v, dst_v, as_v, ad_v, rows_v, ea_v, acc_s, den_s, sem):
    cid = lax.axis_index("c")
    sid = lax.axis_index("s")

    pltpu.sync_copy(src_hbm.at[sid], src_v)
    pltpu.sync_copy(dst_hbm.at[sid], dst_v)
    pltpu.sync_copy(aa_hbm.at[0], as_v)
    pltpu.sync_copy(aa_hbm.at[1], ad_v)

    def _zero_rows(r, _):
        for q in range(DH // 16):
            rows_v[r, pl.ds(16 * q, 16)] = jnp.zeros((16,), jnp.float32)
        return 0
    lax.fori_loop(0, K, _zero_rows, 0)
    for j in range(K // 16):
        ea_v[pl.ds(16 * j, 16)] = jnp.zeros((16,), jnp.float32)
    r0 = sid * ROWS_PER_TILE
    for t in range(5):
        pltpu.sync_copy(rows_v, acc_s.at[pl.ds(r0 + 128 * t, 128)])
        pltpu.sync_copy(ea_v, den_s.at[pl.ds(r0 + 128 * t, 128)])

    plsc.subcore_barrier()

    ebase = sid * PER_TILE
    lane = lax.iota(jnp.int32, 16)
    h_half = h2_hbm.at[cid]

    def _chunk(c, _):
        pltpu.async_copy(h_half.at[src_v.at[c]], rows_v, sem).wait()
        for j in range(K // 16):
            s16 = src_v[c, pl.ds(16 * j, 16)]
            d16 = dst_v[c, pl.ds(16 * j, 16)]
            a = plsc.load_gather(as_v, [s16]) + plsc.load_gather(ad_v, [d16])
            a = jnp.where(a > 0, a, 0.2 * a)
            gid = ebase + c * K + 16 * j + lane
            ea = jnp.where(gid < E_TOTAL, jnp.exp(a), 0.0)
            ea_v[pl.ds(16 * j, 16)] = ea
            for l in range(16):
                s = ea[l]
                r = 16 * j + l
                for q in range(DH // 16):
                    rows_v[r, pl.ds(16 * q, 16)] = rows_v[r, pl.ds(16 * q, 16)] * s
        pltpu.sync_copy(rows_v, acc_s.at[dst_v.at[c]], add=True)
        pltpu.sync_copy(ea_v, den_s.at[dst_v.at[c]], add=True)
        return 0

    lax.fori_loop(0, C, _chunk, 0)

    plsc.subcore_barrier()

    _copy_row_range(acc_s, acc_out.at[cid], r0)
    _copy_row_range(den_s, den_out.at[cid], r0)


@functools.partial(
    pl.kernel,
    out_type=(
        jax.ShapeDtypeStruct((2, NP, DH), jnp.float32),
        jax.ShapeDtypeStruct((2, NP), jnp.float32),
    ),
    mesh=plsc.VectorSubcoreMesh(core_axis_name="c", subcore_axis_name="s"),
    compiler_params=pltpu.CompilerParams(needs_layout_passes=False,
                                         use_tc_tiling_on_sc=False),
    scratch_types=[
        pltpu.VMEM((C, K), jnp.int32),
        pltpu.VMEM((C, K), jnp.int32),
        pltpu.VMEM((N,), jnp.float32),
        pltpu.VMEM((N,), jnp.float32),
        pltpu.VMEM((K, DH), jnp.float32),
        pltpu.VMEM((K,), jnp.float32),
        pltpu.VMEM_SHARED((NP, DH), jnp.float32),
        pltpu.VMEM_SHARED((NP,), jnp.float32),
        pltpu.SemaphoreType.DMA,
    ],
)
def _edge_kernel(h2_hbm, aa_hbm, src_hbm, dst_hbm, acc_out, den_out,
                 src_v, dst_v, as_v, ad_v, rows_v, ea_v, acc_s, den_s, sem):
    _edge_body(h2_hbm, aa_hbm, src_hbm, dst_hbm, acc_out, den_out,
               src_v, dst_v, as_v, ad_v, rows_v, ea_v, acc_s, den_s, sem)


def _stage3_body(acc_ref, den_ref, bias_ref, bnw_ref, bnb_ref, o_ref):
    num = jnp.concatenate([acc_ref[0, :N], acc_ref[1, :N]], axis=1)
    den = den_ref[0, :N]
    val = num / den + bias_ref[...]
    mean = jnp.mean(val, axis=0, keepdims=True)
    ctr = val - mean
    var = jnp.mean(ctr * ctr, axis=0, keepdims=True)
    out = ctr * lax.rsqrt(var + EPS) * bnw_ref[...] + bnb_ref[...]
    o_ref[...] = jnp.maximum(out, 0.0)


def _stage3(acc, den3, bias, bnw, bnb):
    return pl.pallas_call(
        _stage3_body,
        out_shape=jax.ShapeDtypeStruct((N, D), jnp.float32),
    )(acc, den3, bias, bnw, bnb)


def kernel(x, edge_index, edge_attr, W, att_src, att_dst, bias, bn_weight, bn_bias):
    del edge_attr
    ap = jnp.concatenate(
        [att_src[:, None], att_dst[:, None], jnp.zeros((D, 6), jnp.float32)], axis=1)
    h2, aa8 = _stage1(x, W, ap)
    aa = aa8.T[:2]

    loop = jnp.arange(N, dtype=jnp.int32)
    pad = jnp.zeros((E_PAD - E_TOTAL,), jnp.int32)
    src = jnp.concatenate([edge_index[0], loop, pad]).reshape(NT, C, K)
    dst = jnp.concatenate([edge_index[1], loop, pad]).reshape(NT, C, K)

    acc, den = _edge_kernel(h2, aa, src, dst)

    return _stage3(acc, den[:, :, None], bias[None, :],
                   bn_weight[None, :], bn_bias[None, :])


# speedup vs baseline: 38.2965x; 1.4681x over previous
import functools

import jax
import jax.numpy as jnp
from jax import lax
from jax.experimental import pallas as pl
from jax.experimental.pallas import tpu as pltpu, tpu_sc as plsc

N = 10000
D = 128
DH = D // 2
E = 320000
E_TOTAL = E + N
NT = 16
K = 128
C = (E_TOTAL + NT * K - 1) // (NT * K)
E_PAD = NT * C * K
PER_TILE = C * K
EPS = 1e-5
NBUF = 3
assert C % NBUF == 0

NP = 10240
ROWS_PER_TILE = NP // NT


def _stage1_body(x_ref, w_ref, ap_ref, h2_ref, aa8_ref):
    h = jnp.dot(x_ref[...], w_ref[...], preferred_element_type=jnp.float32)
    h2_ref[0] = h[:, :DH]
    h2_ref[1] = h[:, DH:]
    aa8_ref[...] = jnp.dot(h, ap_ref[...], preferred_element_type=jnp.float32)


def _stage1(x, W, ap):
    return pl.pallas_call(
        _stage1_body,
        out_shape=[
            jax.ShapeDtypeStruct((2, N, DH), jnp.float32),
            jax.ShapeDtypeStruct((N, 8), jnp.float32),
        ],
    )(x, W, ap)


def _copy_row_range(src_ref, dst_ref, r0):
    for t in range(5):
        pltpu.sync_copy(src_ref.at[pl.ds(r0 + 128 * t, 128)],
                        dst_ref.at[pl.ds(r0 + 128 * t, 128)])


def _edge_body(h2_hbm, aa_hbm, src_hbm, dst_hbm, acc_out, den_out,
               src_v, dst_v, as_v, ad_v, rows_v, ea_v, acc_s, den_s,
               gsem, ssem):
    cid = lax.axis_index("c")
    sid = lax.axis_index("s")

    pltpu.sync_copy(src_hbm.at[sid], src_v)
    pltpu.sync_copy(dst_hbm.at[sid], dst_v)
    pltpu.sync_copy(aa_hbm.at[0], as_v)
    pltpu.sync_copy(aa_hbm.at[1], ad_v)

    def _zero_rows(r, _):
        for q in range(DH // 16):
            rows_v[0, r, pl.ds(16 * q, 16)] = jnp.zeros((16,), jnp.float32)
        return 0
    lax.fori_loop(0, K, _zero_rows, 0)
    for j in range(K // 16):
        ea_v[0, pl.ds(16 * j, 16)] = jnp.zeros((16,), jnp.float32)
    r0 = sid * ROWS_PER_TILE
    for t in range(5):
        pltpu.sync_copy(rows_v.at[0], acc_s.at[pl.ds(r0 + 128 * t, 128)])
        pltpu.sync_copy(ea_v.at[0], den_s.at[pl.ds(r0 + 128 * t, 128)])

    plsc.subcore_barrier()

    ebase = sid * PER_TILE
    lane = lax.iota(jnp.int32, 16)
    h_half = h2_hbm.at[cid]
    rows = [rows_v.at[b] for b in range(NBUF)]
    eas = [ea_v.at[b] for b in range(NBUF)]
    gsems = [gsem.at[b] for b in range(NBUF)]
    ssems = [ssem.at[b] for b in range(NBUF)]

    def _gather(c, b):
        return pltpu.make_async_copy(h_half.at[src_v.at[c]], rows[b], gsems[b])

    def _scats(c, b):
        return (pltpu.make_async_copy(rows[b], acc_s.at[dst_v.at[c]], ssems[b]),
                pltpu.make_async_copy(eas[b], den_s.at[dst_v.at[c]], ssems[b]))

    _gather(0, 0).start()

    def _outer(i, _):
        for b in range(NBUF):
            c = NBUF * i + b
            bn = (b + 1) % NBUF

            @pl.when(c >= 2)
            def _drain():
                for d in _scats(c - 2, bn):
                    d.wait()

            @pl.when(c + 1 < C)
            def _prefetch():
                _gather(c + 1, bn).start()

            _gather(c, b).wait()

            for j in range(K // 16):
                s16 = src_v[c, pl.ds(16 * j, 16)]
                d16 = dst_v[c, pl.ds(16 * j, 16)]
                a = plsc.load_gather(as_v, [s16]) + plsc.load_gather(ad_v, [d16])
                a = jnp.where(a > 0, a, 0.2 * a)
                gid = ebase + c * K + 16 * j + lane
                ea = jnp.where(gid < E_TOTAL, jnp.exp(a), 0.0)
                eas[b][pl.ds(16 * j, 16)] = ea
                for l in range(16):
                    s = ea[l]
                    r = 16 * j + l
                    for q in range(DH // 16):
                        rows[b][r, pl.ds(16 * q, 16)] = (
                            rows[b][r, pl.ds(16 * q, 16)] * s)
            pltpu.async_copy(rows[b], acc_s.at[dst_v.at[c]], ssems[b], add=True)
            pltpu.async_copy(eas[b], den_s.at[dst_v.at[c]], ssems[b], add=True)
        return 0

    lax.fori_loop(0, C // NBUF, _outer, 0)

    for c in (C - 2, C - 1):
        for d in _scats(c, c % NBUF):
            d.wait()

    plsc.subcore_barrier()

    _copy_row_range(acc_s, acc_out.at[cid], r0)
    _copy_row_range(den_s, den_out.at[cid], r0)


@functools.partial(
    pl.kernel,
    out_type=(
        jax.ShapeDtypeStruct((2, NP, DH), jnp.float32),
        jax.ShapeDtypeStruct((2, NP), jnp.float32),
    ),
    mesh=plsc.VectorSubcoreMesh(core_axis_name="c", subcore_axis_name="s"),
    compiler_params=pltpu.CompilerParams(needs_layout_passes=False,
                                         use_tc_tiling_on_sc=False),
    scratch_types=[
        pltpu.VMEM((C, K), jnp.int32),
        pltpu.VMEM((C, K), jnp.int32),
        pltpu.VMEM((N,), jnp.float32),
        pltpu.VMEM((N,), jnp.float32),
        pltpu.VMEM((NBUF, K, DH), jnp.float32),
        pltpu.VMEM((NBUF, K), jnp.float32),
        pltpu.VMEM_SHARED((NP, DH), jnp.float32),
        pltpu.VMEM_SHARED((NP,), jnp.float32),
        pltpu.SemaphoreType.DMA((NBUF,)),
        pltpu.SemaphoreType.DMA((NBUF,)),
    ],
)
def _edge_kernel(h2_hbm, aa_hbm, src_hbm, dst_hbm, acc_out, den_out,
                 src_v, dst_v, as_v, ad_v, rows_v, ea_v, acc_s, den_s,
                 gsem, ssem):
    _edge_body(h2_hbm, aa_hbm, src_hbm, dst_hbm, acc_out, den_out,
               src_v, dst_v, as_v, ad_v, rows_v, ea_v, acc_s, den_s,
               gsem, ssem)


def _stage3_body(acc_ref, den_ref, bias_ref, bnw_ref, bnb_ref, o_ref):
    num = jnp.concatenate([acc_ref[0, :N], acc_ref[1, :N]], axis=1)
    den = den_ref[0, :N]
    val = num / den + bias_ref[...]
    mean = jnp.mean(val, axis=0, keepdims=True)
    ctr = val - mean
    var = jnp.mean(ctr * ctr, axis=0, keepdims=True)
    out = ctr * lax.rsqrt(var + EPS) * bnw_ref[...] + bnb_ref[...]
    o_ref[...] = jnp.maximum(out, 0.0)


def _stage3(acc, den3, bias, bnw, bnb):
    return pl.pallas_call(
        _stage3_body,
        out_shape=jax.ShapeDtypeStruct((N, D), jnp.float32),
    )(acc, den3, bias, bnw, bnb)


def kernel(x, edge_index, edge_attr, W, att_src, att_dst, bias, bn_weight, bn_bias):
    del edge_attr
    ap = jnp.concatenate(
        [att_src[:, None], att_dst[:, None], jnp.zeros((D, 6), jnp.float32)], axis=1)
    h2, aa8 = _stage1(x, W, ap)
    aa = aa8.T[:2]

    loop = jnp.arange(N, dtype=jnp.int32)
    pad = jnp.zeros((E_PAD - E_TOTAL,), jnp.int32)
    src = jnp.concatenate([edge_index[0], loop, pad]).reshape(NT, C, K)
    dst = jnp.concatenate([edge_index[1], loop, pad]).reshape(NT, C, K)

    acc, den = _edge_kernel(h2, aa, src, dst)

    return _stage3(acc, den[:, :, None], bias[None, :],
                   bn_weight[None, :], bn_bias[None, :])


# speedup vs baseline: 57.1657x; 1.4927x over previous
import functools

import jax
import jax.numpy as jnp
from jax import lax
from jax.experimental import pallas as pl
from jax.experimental.pallas import tpu as pltpu, tpu_sc as plsc

N = 10000
D = 128
DH = D // 2
E = 320000
NT = 16
K = 128
CR = E // K
CR_LO = CR // NT
NX = CR - NT * CR_LO
CSZ = CR_LO + 1
NBUF = 3
CT = ((CSZ + NBUF) // NBUF) * NBUF
EPS = 1e-5

NP = 10240
ROWS_PER_TILE = NP // NT
NB_NODE = ROWS_PER_TILE // K


def _stage1_body(x_ref, w_ref, ap_ref, h2_ref, aa_ref):
    h = jnp.dot(x_ref[...], w_ref[...], preferred_element_type=jnp.float32)
    zpad = jnp.zeros((NP - N, DH), jnp.float32)
    h2_ref[0] = jnp.concatenate([h[:, :DH], zpad], axis=0)
    h2_ref[1] = jnp.concatenate([h[:, DH:], zpad], axis=0)
    aap = lax.dot_general(ap_ref[...], h, (((0,), (1,)), ((), ())),
                          preferred_element_type=jnp.float32)
    aa_ref[...] = jnp.concatenate(
        [aap, jnp.zeros((8, NP - N), jnp.float32)], axis=1)


def _stage1(x, W, ap):
    return pl.pallas_call(
        _stage1_body,
        out_shape=[
            jax.ShapeDtypeStruct((2, NP, DH), jnp.float32),
            jax.ShapeDtypeStruct((8, NP), jnp.float32),
        ],
    )(x, W, ap)


def _edge_body(h2_hbm, aa_hbm, e2_hbm, acc_out,
               src_v, dst_v, as_v, ad_v, rows_v, ea_v, idx_v,
               acc_s, den_s, gsem, ssem):
    cid = lax.axis_index("c")
    sid = lax.axis_index("s")

    n_c = jnp.where(sid < NX, CR_LO + 1, CR_LO)
    cb = sid * CR_LO + jnp.minimum(sid, NX)
    last = jnp.minimum(cb + CR_LO, CR - 1)
    pltpu.sync_copy(e2_hbm.at[0].at[pl.ds(cb, CR_LO)], src_v.at[pl.ds(0, CR_LO)])
    pltpu.sync_copy(e2_hbm.at[1].at[pl.ds(cb, CR_LO)], dst_v.at[pl.ds(0, CR_LO)])
    pltpu.sync_copy(e2_hbm.at[0].at[last], src_v.at[CR_LO])
    pltpu.sync_copy(e2_hbm.at[1].at[last], dst_v.at[CR_LO])

    pltpu.sync_copy(aa_hbm.at[0], as_v)
    pltpu.sync_copy(aa_hbm.at[1], ad_v)

    def _zero_rows(r, _):
        for q in range(DH // 16):
            rows_v[0, r, pl.ds(16 * q, 16)] = jnp.zeros((16,), jnp.float32)
        return 0
    lax.fori_loop(0, K, _zero_rows, 0)
    for j in range(K // 16):
        ea_v[0, pl.ds(16 * j, 16)] = jnp.zeros((16,), jnp.float32)
    r0 = sid * ROWS_PER_TILE
    for t in range(NB_NODE):
        pltpu.sync_copy(rows_v.at[0], acc_s.at[pl.ds(r0 + K * t, K)])
        pltpu.sync_copy(ea_v.at[0], den_s.at[pl.ds(r0 + K * t, K)])

    plsc.subcore_barrier()

    lane = lax.iota(jnp.int32, 16)
    h_half = h2_hbm.at[cid]

    def _gather(c, b):
        cc = jnp.minimum(c, CSZ - 1)
        return pltpu.make_async_copy(h_half.at[src_v.at[cc]], rows_v.at[b],
                                     gsem.at[b])

    def _scats(c, b):
        cc = jnp.minimum(c, CSZ - 1)
        return (pltpu.make_async_copy(rows_v.at[b], acc_s.at[dst_v.at[cc]],
                                      ssem.at[b]),
                pltpu.make_async_copy(ea_v.at[b], den_s.at[dst_v.at[cc]],
                                      ssem.at[b]))

    _gather(0, 0).start()

    def _chunk(c, _):
        b = lax.rem(c, NBUF)
        bn = lax.rem(c + 1, NBUF)

        @pl.when(c >= 2)
        def _drain():
            for d in _scats(c - 2, bn):
                d.wait()

        @pl.when(c + 1 < CT)
        def _prefetch():
            _gather(c + 1, bn).start()

        _gather(c, b).wait()

        cc = jnp.minimum(c, CSZ - 1)
        valid = c < n_c
        for j in range(K // 16):
            s16 = src_v[cc, pl.ds(16 * j, 16)]
            d16 = dst_v[cc, pl.ds(16 * j, 16)]
            a = plsc.load_gather(as_v, [s16]) + plsc.load_gather(ad_v, [d16])
            a = jnp.where(a > 0, a, 0.2 * a)
            ea = jnp.where(valid, jnp.exp(a), 0.0)
            ea_v[b, pl.ds(16 * j, 16)] = ea
            for l in range(16):
                s = ea[l]
                r = 16 * j + l
                for q in range(DH // 16):
                    rows_v[b, r, pl.ds(16 * q, 16)] = (
                        rows_v[b, r, pl.ds(16 * q, 16)] * s)

        pltpu.async_copy(rows_v.at[b], acc_s.at[dst_v.at[cc]], ssem.at[b],
                         add=True)
        pltpu.async_copy(ea_v.at[b], den_s.at[dst_v.at[cc]], ssem.at[b],
                         add=True)
        return 0

    lax.fori_loop(0, CT, _chunk, 0)
    for c in (CT - 2, CT - 1):
        for d in _scats(c, c % NBUF):
            d.wait()

    def _selfloop(t, _):
        nb = r0 + K * t
        pltpu.sync_copy(h_half.at[pl.ds(nb, K)], rows_v.at[0])
        for j in range(K // 16):
            node = nb + 16 * j + lane
            a = as_v[pl.ds(nb + 16 * j, 16)] + ad_v[pl.ds(nb + 16 * j, 16)]
            a = jnp.where(a > 0, a, 0.2 * a)
            ea = jnp.where(node < N, jnp.exp(a), 0.0)
            ea_v[0, pl.ds(16 * j, 16)] = ea
            idx_v[pl.ds(16 * j, 16)] = node
            for l in range(16):
                s = ea[l]
                r = 16 * j + l
                for q in range(DH // 16):
                    rows_v[0, r, pl.ds(16 * q, 16)] = (
                        rows_v[0, r, pl.ds(16 * q, 16)] * s)
        pltpu.sync_copy(rows_v.at[0], acc_s.at[idx_v], add=True)
        pltpu.sync_copy(ea_v.at[0], den_s.at[idx_v], add=True)
        return 0

    lax.fori_loop(0, NB_NODE, _selfloop, 0)

    plsc.subcore_barrier()

    out_half = acc_out.at[cid]

    def _normalize(t, _):
        nb = r0 + K * t
        pltpu.sync_copy(acc_s.at[pl.ds(nb, K)], rows_v.at[0])
        pltpu.sync_copy(den_s.at[pl.ds(nb, K)], ea_v.at[0])
        for j in range(K // 16):
            inv = 1.0 / ea_v[0, pl.ds(16 * j, 16)]
            for l in range(16):
                s = inv[l]
                r = 16 * j + l
                for q in range(DH // 16):
                    rows_v[0, r, pl.ds(16 * q, 16)] = (
                        rows_v[0, r, pl.ds(16 * q, 16)] * s)
        pltpu.sync_copy(rows_v.at[0], out_half.at[pl.ds(nb, K)])
        return 0

    lax.fori_loop(0, NB_NODE, _normalize, 0)


@functools.partial(
    pl.kernel,
    out_type=jax.ShapeDtypeStruct((2, NP, DH), jnp.float32),
    mesh=plsc.VectorSubcoreMesh(core_axis_name="c", subcore_axis_name="s"),
    compiler_params=pltpu.CompilerParams(needs_layout_passes=False,
                                         use_tc_tiling_on_sc=False),
    scratch_types=[
        pltpu.VMEM((CSZ, K), jnp.int32),
        pltpu.VMEM((CSZ, K), jnp.int32),
        pltpu.VMEM((NP,), jnp.float32),
        pltpu.VMEM((NP,), jnp.float32),
        pltpu.VMEM((NBUF, K, DH), jnp.float32),
        pltpu.VMEM((NBUF, K), jnp.float32),
        pltpu.VMEM((K,), jnp.int32),
        pltpu.VMEM_SHARED((NP, DH), jnp.float32),
        pltpu.VMEM_SHARED((NP,), jnp.float32),
        pltpu.SemaphoreType.DMA((NBUF,)),
        pltpu.SemaphoreType.DMA((NBUF,)),
    ],
)
def _edge_kernel(h2_hbm, aa_hbm, e2_hbm, acc_out,
                 src_v, dst_v, as_v, ad_v, rows_v, ea_v, idx_v,
                 acc_s, den_s, gsem, ssem):
    _edge_body(h2_hbm, aa_hbm, e2_hbm, acc_out,
               src_v, dst_v, as_v, ad_v, rows_v, ea_v, idx_v,
               acc_s, den_s, gsem, ssem)


def _stage3_body(acc_ref, bias_ref, bnw_ref, bnb_ref, o_ref):
    val = jnp.concatenate([acc_ref[0, :N], acc_ref[1, :N]], axis=1)
    val = val + bias_ref[...]
    mean = jnp.mean(val, axis=0, keepdims=True)
    ctr = val - mean
    var = jnp.mean(ctr * ctr, axis=0, keepdims=True)
    out = ctr * lax.rsqrt(var + EPS) * bnw_ref[...] + bnb_ref[...]
    o_ref[...] = jnp.maximum(out, 0.0)


def _stage3(acc, bias, bnw, bnb):
    return pl.pallas_call(
        _stage3_body,
        out_shape=jax.ShapeDtypeStruct((N, D), jnp.float32),
    )(acc, bias, bnw, bnb)


def kernel(x, edge_index, edge_attr, W, att_src, att_dst, bias, bn_weight, bn_bias):
    del edge_attr
    ap = jnp.concatenate(
        [att_src[:, None], att_dst[:, None], jnp.zeros((D, 6), jnp.float32)], axis=1)
    h2, aa = _stage1(x, W, ap)
    e2 = edge_index.reshape(2, CR, K)
    acc = _edge_kernel(h2, aa, e2)
    return _stage3(acc, bias[None, :], bn_weight[None, :], bn_bias[None, :])


# speedup vs baseline: 57.1995x; 1.0006x over previous
import functools

import jax
import jax.numpy as jnp
from jax import lax
from jax.experimental import pallas as pl
from jax.experimental.pallas import tpu as pltpu, tpu_sc as plsc

N = 10000
D = 128
DH = D // 2
E = 320000
NT = 16
K = 128
CR = E // K
CR_LO = CR // NT
NX = CR - NT * CR_LO
CSZ = CR_LO + 1
NBUF = 3
CT = ((CSZ + NBUF) // NBUF) * NBUF
EPS = 1e-5

NP = 10240
ROWS_PER_TILE = NP // NT
NB_NODE = ROWS_PER_TILE // K


def _stage1_body(x_ref, w_ref, ap_ref, h2_ref, aa_ref):
    h = jnp.dot(x_ref[...], w_ref[...], preferred_element_type=jnp.float32)
    zpad = jnp.zeros((NP - N, DH), jnp.float32)
    h2_ref[0] = jnp.concatenate([h[:, :DH], zpad], axis=0)
    h2_ref[1] = jnp.concatenate([h[:, DH:], zpad], axis=0)
    aap = lax.dot_general(ap_ref[...], h, (((0,), (1,)), ((), ())),
                          preferred_element_type=jnp.float32)
    aa_ref[...] = jnp.concatenate(
        [aap, jnp.zeros((8, NP - N), jnp.float32)], axis=1)


def _stage1(x, W, ap):
    return pl.pallas_call(
        _stage1_body,
        out_shape=[
            jax.ShapeDtypeStruct((2, NP, DH), jnp.float32),
            jax.ShapeDtypeStruct((8, NP), jnp.float32),
        ],
    )(x, W, ap)


def _edge_body(h2_hbm, aa_hbm, e2_hbm, acc_out,
               src_v, dst_v, dst2_v, as_v, ad_v, rows_v, ea_v, idx_v,
               acc_s, den_s, gsem, ssem):
    cid = lax.axis_index("c")
    sid = lax.axis_index("s")

    n_c = jnp.where(sid < NX, CR_LO + 1, CR_LO)
    cb = sid * CR_LO + jnp.minimum(sid, NX)
    last = jnp.minimum(cb + CR_LO, CR - 1)
    pltpu.sync_copy(e2_hbm.at[0].at[pl.ds(cb * K, CR_LO * K)],
                    src_v.at[pl.ds(0, CR_LO * K)])
    pltpu.sync_copy(e2_hbm.at[1].at[pl.ds(cb * K, CR_LO * K)],
                    dst_v.at[pl.ds(0, CR_LO * K)])
    pltpu.sync_copy(e2_hbm.at[0].at[pl.ds(last * K, K)],
                    src_v.at[pl.ds(CR_LO * K, K)])
    pltpu.sync_copy(e2_hbm.at[1].at[pl.ds(last * K, K)],
                    dst_v.at[pl.ds(CR_LO * K, K)])

    pltpu.sync_copy(aa_hbm.at[0], as_v)
    pltpu.sync_copy(aa_hbm.at[1], ad_v)

    def _zero_rows(r, _):
        for q in range(DH // 16):
            rows_v[0, r, pl.ds(16 * q, 16)] = jnp.zeros((16,), jnp.float32)
        return 0
    lax.fori_loop(0, K, _zero_rows, 0)
    for j in range(K // 16):
        ea_v[0, pl.ds(16 * j, 16)] = jnp.zeros((16,), jnp.float32)
    r0 = sid * ROWS_PER_TILE
    for t in range(NB_NODE):
        pltpu.sync_copy(rows_v.at[0], acc_s.at[pl.ds(r0 + K * t, K)])
        pltpu.sync_copy(ea_v.at[0], den_s.at[pl.ds(r0 + K * t, K)])

    plsc.subcore_barrier()

    lane = lax.iota(jnp.int32, 16)
    h_half = h2_hbm.at[cid]

    def _gather(c, b):
        cc = jnp.minimum(c, CSZ - 1)
        return pltpu.make_async_copy(h_half.at[src_v.at[pl.ds(cc * K, K)]],
                                     rows_v.at[b], gsem.at[b])

    def _scats(c, b):
        return (pltpu.make_async_copy(rows_v.at[b], acc_s.at[dst2_v.at[b]],
                                      ssem.at[b]),
                pltpu.make_async_copy(ea_v.at[b], den_s.at[dst2_v.at[b]],
                                      ssem.at[b]))

    _gather(0, 0).start()

    def _chunk(c, _):
        b = lax.rem(c, NBUF)
        bn = lax.rem(c + 1, NBUF)

        @pl.when(c >= 2)
        def _drain():
            for d in _scats(c - 2, bn):
                d.wait()

        @pl.when(c + 1 < CT)
        def _prefetch():
            _gather(c + 1, bn).start()

        _gather(c, b).wait()

        cc = jnp.minimum(c, CSZ - 1)
        valid = c < n_c
        for j in range(K // 16):
            s16 = src_v[pl.ds(cc * K + 16 * j, 16)]
            d16 = dst_v[pl.ds(cc * K + 16 * j, 16)]
            dst2_v[b, pl.ds(16 * j, 16)] = d16
            a = plsc.load_gather(as_v, [s16]) + plsc.load_gather(ad_v, [d16])
            a = jnp.where(a > 0, a, 0.2 * a)
            ea = jnp.where(valid, jnp.exp(a), 0.0)
            ea_v[b, pl.ds(16 * j, 16)] = ea
            for l in range(16):
                s = ea[l]
                r = 16 * j + l
                for q in range(DH // 16):
                    rows_v[b, r, pl.ds(16 * q, 16)] = (
                        rows_v[b, r, pl.ds(16 * q, 16)] * s)

        pltpu.async_copy(rows_v.at[b], acc_s.at[dst2_v.at[b]], ssem.at[b],
                         add=True)
        pltpu.async_copy(ea_v.at[b], den_s.at[dst2_v.at[b]], ssem.at[b],
                         add=True)
        return 0

    lax.fori_loop(0, CT, _chunk, 0)
    for c in (CT - 2, CT - 1):
        for d in _scats(c, c % NBUF):
            d.wait()

    def _selfloop(t, _):
        nb = r0 + K * t
        pltpu.sync_copy(h_half.at[pl.ds(nb, K)], rows_v.at[0])
        for j in range(K // 16):
            node = nb + 16 * j + lane
            a = as_v[pl.ds(nb + 16 * j, 16)] + ad_v[pl.ds(nb + 16 * j, 16)]
            a = jnp.where(a > 0, a, 0.2 * a)
            ea = jnp.where(node < N, jnp.exp(a), 0.0)
            ea_v[0, pl.ds(16 * j, 16)] = ea
            idx_v[pl.ds(16 * j, 16)] = node
            for l in range(16):
                s = ea[l]
                r = 16 * j + l
                for q in range(DH // 16):
                    rows_v[0, r, pl.ds(16 * q, 16)] = (
                        rows_v[0, r, pl.ds(16 * q, 16)] * s)
        pltpu.sync_copy(rows_v.at[0], acc_s.at[idx_v], add=True)
        pltpu.sync_copy(ea_v.at[0], den_s.at[idx_v], add=True)
        return 0

    lax.fori_loop(0, NB_NODE, _selfloop, 0)

    plsc.subcore_barrier()

    out_half = acc_out.at[cid]

    def _normalize(t, _):
        nb = r0 + K * t
        pltpu.sync_copy(acc_s.at[pl.ds(nb, K)], rows_v.at[0])
        pltpu.sync_copy(den_s.at[pl.ds(nb, K)], ea_v.at[0])
        for j in range(K // 16):
            inv = 1.0 / ea_v[0, pl.ds(16 * j, 16)]
            for l in range(16):
                s = inv[l]
                r = 16 * j + l
                for q in range(DH // 16):
                    rows_v[0, r, pl.ds(16 * q, 16)] = (
                        rows_v[0, r, pl.ds(16 * q, 16)] * s)
        pltpu.sync_copy(rows_v.at[0], out_half.at[pl.ds(nb, K)])
        return 0

    lax.fori_loop(0, NB_NODE, _normalize, 0)


@functools.partial(
    pl.kernel,
    out_type=jax.ShapeDtypeStruct((2, NP, DH), jnp.float32),
    mesh=plsc.VectorSubcoreMesh(core_axis_name="c", subcore_axis_name="s"),
    compiler_params=pltpu.CompilerParams(needs_layout_passes=False,
                                         use_tc_tiling_on_sc=False),
    scratch_types=[
        pltpu.VMEM((CSZ * K,), jnp.int32),
        pltpu.VMEM((CSZ * K,), jnp.int32),
        pltpu.VMEM((NBUF, K), jnp.int32),
        pltpu.VMEM((NP,), jnp.float32),
        pltpu.VMEM((NP,), jnp.float32),
        pltpu.VMEM((NBUF, K, DH), jnp.float32),
        pltpu.VMEM((NBUF, K), jnp.float32),
        pltpu.VMEM((K,), jnp.int32),
        pltpu.VMEM_SHARED((NP, DH), jnp.float32),
        pltpu.VMEM_SHARED((NP,), jnp.float32),
        pltpu.SemaphoreType.DMA((NBUF,)),
        pltpu.SemaphoreType.DMA((NBUF,)),
    ],
)
def _edge_kernel(h2_hbm, aa_hbm, e2_hbm, acc_out,
                 src_v, dst_v, dst2_v, as_v, ad_v, rows_v, ea_v, idx_v,
                 acc_s, den_s, gsem, ssem):
    _edge_body(h2_hbm, aa_hbm, e2_hbm, acc_out,
               src_v, dst_v, dst2_v, as_v, ad_v, rows_v, ea_v, idx_v,
               acc_s, den_s, gsem, ssem)


def _stage3_body(acc_ref, bias_ref, bnw_ref, bnb_ref, o_ref):
    val = jnp.concatenate([acc_ref[0, :N], acc_ref[1, :N]], axis=1)
    val = val + bias_ref[...]
    mean = jnp.mean(val, axis=0, keepdims=True)
    ctr = val - mean
    var = jnp.mean(ctr * ctr, axis=0, keepdims=True)
    out = ctr * lax.rsqrt(var + EPS) * bnw_ref[...] + bnb_ref[...]
    o_ref[...] = jnp.maximum(out, 0.0)


def _stage3(acc, bias, bnw, bnb):
    return pl.pallas_call(
        _stage3_body,
        out_shape=jax.ShapeDtypeStruct((N, D), jnp.float32),
    )(acc, bias, bnw, bnb)


def kernel(x, edge_index, edge_attr, W, att_src, att_dst, bias, bn_weight, bn_bias):
    del edge_attr
    ap = jnp.concatenate(
        [att_src[:, None], att_dst[:, None], jnp.zeros((D, 6), jnp.float32)], axis=1)
    h2, aa = _stage1(x, W, ap)
    acc = _edge_kernel(h2, aa, edge_index)
    return _stage3(acc, bias[None, :], bn_weight[None, :], bn_bias[None, :])
